# baseline (speedup 1.0000x reference)
import functools

import jax
import jax.numpy as jnp
from jax import lax
from jax.experimental import pallas as pl
from jax.experimental.pallas import tpu as pltpu
from jax.experimental.pallas import tpu_sc as plsc

_N_FEATURES = 12
_N_IMAGES = 100000
_BATCH = 16384
_CHUNK = 128

_info = plsc.get_sparse_core_info()
_NC, _NS = _info.num_cores, _info.num_subcores
_NW = _NC * _NS
_B_PER_W = _BATCH // _NW
_N_CHUNKS = _B_PER_W // _CHUNK

_mesh = plsc.VectorSubcoreMesh(core_axis_name="c", subcore_axis_name="s")


@functools.partial(
    pl.kernel,
    mesh=_mesh,
    out_type=jax.ShapeDtypeStruct((_N_FEATURES, _BATCH), jnp.float32),
    compiler_params=pltpu.CompilerParams(use_tc_tiling_on_sc=False),
    scratch_types=[
        [pltpu.VMEM((_CHUNK,), jnp.int32) for _ in range(_N_CHUNKS)],
        [[pltpu.VMEM((_CHUNK,), jnp.float32) for _ in range(_N_FEATURES)]
         for _ in range(_N_CHUNKS)],
        pltpu.VMEM((_N_CHUNKS * _N_FEATURES * _CHUNK,), jnp.float32),
        [pltpu.SemaphoreType.DMA for _ in range(_N_CHUNKS)],
        pltpu.SemaphoreType.DMA,
        pltpu.SemaphoreType.DMA,
    ],
)
def _gather_cols(idx_hbm, tab_t_hbm, out_hbm, idx_bufs, col_sets,
                 drain_buf, sems, sem_i, sem_w):
    wid = lax.axis_index("s") * _NC + lax.axis_index("c")

    def drain(sem, buf, n):
        pltpu.make_async_copy(
            tab_t_hbm.at[0].at[pl.ds(0, n * _CHUNK)],
            buf.at[pl.ds(0, n * _CHUNK)],
            sem,
        ).wait()

    idx_copies = []
    for g in range(_N_CHUNKS):
        chunk = wid * _N_CHUNKS + g
        idx_copies.append(
            pltpu.async_copy(
                idx_hbm.at[pl.ds(chunk * _CHUNK, _CHUNK)], idx_bufs[g], sem_i
            )
        )

    for g in range(_N_CHUNKS):
        idx_copies[g].wait()
        for c in range(_N_FEATURES):
            pltpu.async_copy(
                tab_t_hbm.at[c].at[idx_bufs[g]], col_sets[g][c], sems[g]
            )

    for g in range(_N_CHUNKS):
        chunk = wid * _N_CHUNKS + g
        drain(sems[g], drain_buf, _N_FEATURES)
        for c in range(_N_FEATURES):
            pltpu.async_copy(
                col_sets[g][c],
                out_hbm.at[c].at[pl.ds(chunk * _CHUNK, _CHUNK)],
                sem_w,
            )

    drain(sem_w, drain_buf, _N_CHUNKS * _N_FEATURES)


def kernel(image_inds, prf_params, prf_model_index, labels_table):
    del prf_params, prf_model_index
    out_t = _gather_cols(image_inds.astype(jnp.int32), labels_table.T)
    features = out_t.T
    feature_inds_defined = jnp.ones((_N_FEATURES,), dtype=bool)
    return (features, feature_inds_defined)


# speedup vs baseline: 1.0030x; 1.0030x over previous
import functools

import jax
import jax.numpy as jnp
from jax import lax
from jax.experimental import pallas as pl
from jax.experimental.pallas import tpu as pltpu
from jax.experimental.pallas import tpu_sc as plsc

_N_FEATURES = 12
_N_IMAGES = 100000
_BATCH = 16384
_CHUNK = 128

_info = plsc.get_sparse_core_info()
_NC, _NS = _info.num_cores, _info.num_subcores
_NW = _NC * _NS
_B_PER_W = _BATCH // _NW
_N_CHUNKS = _B_PER_W // _CHUNK

_mesh = plsc.VectorSubcoreMesh(core_axis_name="c", subcore_axis_name="s")


@functools.partial(
    pl.kernel,
    mesh=_mesh,
    out_type=jax.ShapeDtypeStruct((_N_FEATURES, _BATCH), jnp.float32),
    compiler_params=pltpu.CompilerParams(use_tc_tiling_on_sc=False),
    scratch_types=[
        [pltpu.VMEM((_CHUNK,), jnp.int32) for _ in range(_N_CHUNKS)],
        [[pltpu.VMEM((_CHUNK,), jnp.float32) for _ in range(_N_FEATURES)]
         for _ in range(_N_CHUNKS)],
        pltpu.VMEM((_N_CHUNKS * _N_FEATURES * _CHUNK,), jnp.float32),
        [pltpu.SemaphoreType.DMA for _ in range(_N_CHUNKS)],
        pltpu.SemaphoreType.DMA,
    ],
)
def _gather_cols(idx_hbm, tab_t_hbm, out_hbm, idx_bufs, col_sets,
                 drain_buf, sems, sem_w):
    wid = lax.axis_index("s") * _NC + lax.axis_index("c")

    def drain(sem, buf, n):
        pltpu.make_async_copy(
            tab_t_hbm.at[0].at[pl.ds(0, n * _CHUNK)],
            buf.at[pl.ds(0, n * _CHUNK)],
            sem,
        ).wait()

    idx_copies = []
    for g in range(_N_CHUNKS):
        chunk = wid * _N_CHUNKS + g
        idx_copies.append(
            pltpu.async_copy(
                idx_hbm.at[pl.ds(chunk * _CHUNK, _CHUNK)], idx_bufs[g],
                sems[g],
            )
        )

    for g in range(_N_CHUNKS):
        idx_copies[g].wait()
        for c in range(_N_FEATURES):
            pltpu.async_copy(
                tab_t_hbm.at[c].at[idx_bufs[g]], col_sets[g][c], sems[g]
            )

    for g in range(_N_CHUNKS):
        chunk = wid * _N_CHUNKS + g
        drain(sems[g], drain_buf, _N_FEATURES)
        for c in range(_N_FEATURES):
            pltpu.async_copy(
                col_sets[g][c],
                out_hbm.at[c].at[pl.ds(chunk * _CHUNK, _CHUNK)],
                sem_w,
            )

    drain(sem_w, drain_buf, _N_CHUNKS * _N_FEATURES)


def kernel(image_inds, prf_params, prf_model_index, labels_table):
    del prf_params, prf_model_index
    out_t = _gather_cols(image_inds.astype(jnp.int32), labels_table.T)
    features = out_t.T
    feature_inds_defined = jnp.ones((_N_FEATURES,), dtype=bool)
    return (features, feature_inds_defined)
